# baseline (speedup 1.0000x reference)
import functools

import jax
import jax.numpy as jnp
from jax import lax
from jax.experimental import pallas as pl
from jax.experimental.pallas import tpu as pltpu
from jax.experimental.pallas import tpu_sc as plsc

_N = 100000
_OBS_D = 17
_ACT_D = 6
_NW = 32
_ROWS = 3136


def _sc_copy_body(obs_hbm, act_hbm, obs_out, act_out,
                  obs_v, act_v, sem_obs, sem_act):
    c = lax.axis_index("c")
    s = lax.axis_index("s")
    wid = s * 2 + c
    base = jnp.minimum(wid * _ROWS, _N - _ROWS)

    c_obs = pltpu.async_copy(obs_hbm.at[pl.ds(base, _ROWS), :], obs_v, sem_obs)
    c_act = pltpu.async_copy(act_hbm.at[pl.ds(base, _ROWS), :], act_v, sem_act)
    c_obs.wait()
    o_obs = pltpu.async_copy(obs_v, obs_out.at[pl.ds(base, _ROWS), :], sem_obs)
    c_act.wait()
    o_act = pltpu.async_copy(act_v, act_out.at[pl.ds(base, _ROWS), :], sem_act)
    o_obs.wait()
    o_act.wait()


def kernel(obs_table, act_table):
    k = functools.partial(
        pl.kernel,
        out_type=(
            jax.ShapeDtypeStruct((_N, _OBS_D), jnp.float32),
            jax.ShapeDtypeStruct((_N, _ACT_D), jnp.float32),
        ),
        mesh=plsc.VectorSubcoreMesh(core_axis_name="c", subcore_axis_name="s"),
        compiler_params=pltpu.CompilerParams(use_tc_tiling_on_sc=False),
        scratch_types=[
            pltpu.VMEM((_ROWS, _OBS_D), jnp.float32),
            pltpu.VMEM((_ROWS, _ACT_D), jnp.float32),
            pltpu.SemaphoreType.DMA,
            pltpu.SemaphoreType.DMA,
        ],
    )(_sc_copy_body)
    return k(obs_table, act_table)
